# baseline (speedup 1.0000x reference)
import functools

import jax
import jax.numpy as jnp
from jax import lax
from jax.experimental import pallas as pl
from jax.experimental.pallas import tpu as pltpu
from jax.experimental.pallas import tpu_sc as plsc

N = 10000
NP = 10240
E = 320000
LATENT = 128
MAT = 16
OUT = 128

NC = 2
NS = 16
NW = NC * NS
EPW = E // NW
KD = 80
CHUNKS_D = EPW // KD
KE = 112
FAST_CID = 1
CEF = 120
CES = 60
CE_TOT = CEF + CES
E_PAD = NS * CE_TOT * KE
ROWS_PER_TILE = NP // NS

_mesh = plsc.VectorSubcoreMesh(core_axis_name="c", subcore_axis_name="s",
                               num_cores=NC, num_subcores=NS)


@functools.partial(
    pl.kernel,
    out_type=jax.ShapeDtypeStruct((NC, NP, LATENT), jnp.float32),
    mesh=_mesh,
    scratch_types=[
        pltpu.VMEM((CHUNKS_D, KD), jnp.int32),
        pltpu.VMEM((KD, LATENT), jnp.float32),
        pltpu.VMEM_SHARED((NP, LATENT), jnp.float32),
    ],
)
def _deg_kernel(dst_hbm, ones_hbm, zeros_hbm, out_hbm, idx_v, ones_v, acc_sh):
    cid = lax.axis_index("c")
    sid = lax.axis_index("s")
    wid = cid * NS + sid
    pltpu.sync_copy(dst_hbm.at[wid], idx_v)
    pltpu.sync_copy(ones_hbm, ones_v)
    pltpu.sync_copy(
        zeros_hbm.at[pl.ds(sid * ROWS_PER_TILE, ROWS_PER_TILE)],
        acc_sh.at[pl.ds(sid * ROWS_PER_TILE, ROWS_PER_TILE)],
    )
    plsc.subcore_barrier()

    def body(c, carry):
        pltpu.sync_copy(ones_v, acc_sh.at[idx_v.at[c]], add=True)
        return carry

    lax.fori_loop(0, CHUNKS_D, body, 0)
    plsc.subcore_barrier()
    pltpu.sync_copy(
        acc_sh.at[pl.ds(sid * ROWS_PER_TILE, ROWS_PER_TILE)],
        out_hbm.at[cid, pl.ds(sid * ROWS_PER_TILE, ROWS_PER_TILE)],
    )


@functools.partial(
    pl.kernel,
    out_type=jax.ShapeDtypeStruct((NC, NP, LATENT), jnp.float32),
    mesh=_mesh,
    scratch_types=[
        pltpu.VMEM((5, 2, KE), jnp.int32),
        pltpu.VMEM((3, KE, LATENT), jnp.float32),
        pltpu.VMEM_SHARED((NP, LATENT), jnp.float32),
    ] + [pltpu.SemaphoreType.DMA] * 11,
)
def _edge_kernel(hs_hbm, idx_hbm, zeros_hbm, out_hbm, idx_v, rows_v, acc_sh,
                 isem0, isem1, isem2, isem3, isem4,
                 gsem0, gsem1, gsem2, ssem0, ssem1, ssem2):
    cid = lax.axis_index("c")
    sid = lax.axis_index("s")
    wid = cid * NS + sid
    cnt = jnp.where(cid == FAST_CID, CEF, CES)
    isems = (isem0, isem1, isem2, isem3, isem4)
    gsems = (gsem0, gsem1, gsem2)
    ssems = (ssem0, ssem1, ssem2)

    for i in range(5):
        pltpu.async_copy(idx_hbm.at[wid, i], idx_v.at[i], isems[i])
    pltpu.sync_copy(
        zeros_hbm.at[pl.ds(sid * ROWS_PER_TILE, ROWS_PER_TILE)],
        acc_sh.at[pl.ds(sid * ROWS_PER_TILE, ROWS_PER_TILE)],
    )
    plsc.subcore_barrier()
    for c0 in (0, 1):
        pltpu.make_async_copy(
            idx_hbm.at[wid, c0], idx_v.at[c0], isems[c0]).wait()
        pltpu.async_copy(hs_hbm.at[idx_v.at[c0, 0]], rows_v.at[c0],
                         gsems[c0])

    NSTEP = 15

    def body(g, carry):
        base = g * NSTEP
        for st in range(NSTEP):
            r = st % 3
            rm1 = (st - 1) % 3
            rp2 = (st + 2) % 3
            i = st % 5
            im1 = (st - 1) % 5
            ip2 = (st + 2) % 5
            ip4 = (st + 4) % 5
            c = base + st

            pltpu.make_async_copy(
                hs_hbm.at[idx_v.at[i, 0]], rows_v.at[r], gsems[r]).wait()
            pltpu.async_copy(
                rows_v.at[r], acc_sh.at[idx_v.at[i, 1]], ssems[r], add=True)

            @pl.when(c >= 1)
            def _():
                pltpu.make_async_copy(
                    rows_v.at[rm1], acc_sh.at[idx_v.at[im1, 1]],
                    ssems[rm1]).wait()

            @pl.when((c >= 1) & (c + 4 < cnt))
            def _():
                pltpu.async_copy(
                    idx_hbm.at[wid, c + 4], idx_v.at[ip4], isems[ip4])

            @pl.when(c + 2 < cnt)
            def _():
                pltpu.make_async_copy(
                    idx_hbm.at[wid, c + 2], idx_v.at[ip2], isems[ip2]).wait()
                pltpu.async_copy(
                    hs_hbm.at[idx_v.at[ip2, 0]], rows_v.at[rp2], gsems[rp2])
        return carry

    lax.fori_loop(0, cnt // NSTEP, body, 0)
    lr = (CEF - 1) % 3
    li = (CEF - 1) % 5
    pltpu.make_async_copy(
        rows_v.at[lr], acc_sh.at[idx_v.at[li, 1]], ssems[lr]).wait()
    plsc.subcore_barrier()
    pltpu.sync_copy(
        acc_sh.at[pl.ds(sid * ROWS_PER_TILE, ROWS_PER_TILE)],
        out_hbm.at[cid, pl.ds(sid * ROWS_PER_TILE, ROWS_PER_TILE)],
    )


RB = 1024
GRID = NP // RB


def _dis(da_ref, db_ref):
    deg = 1.0 + da_ref[:, 0:1] + db_ref[:, 0:1]
    return lax.rsqrt(deg)


def _l1_body(z_ref, mp_ref, da_ref, db_ref, w1z_ref, w1m_ref, o_ref):
    h = jnp.dot(z_ref[...], w1z_ref[...], preferred_element_type=jnp.float32)
    h = h + jnp.dot(mp_ref[...], w1m_ref[...],
                    preferred_element_type=jnp.float32)
    o_ref[...] = h * _dis(da_ref, db_ref)


def _l2_body(aa_ref, ab_ref, hs_ref, da_ref, db_ref, b1_ref, w2_ref, o_ref):
    dis = _dis(da_ref, db_ref)
    x = dis * (aa_ref[...] + ab_ref[...] + hs_ref[...]) + b1_ref[...]
    x = jnp.maximum(x, 0.0)
    o_ref[...] = jnp.dot(x, w2_ref[...],
                         preferred_element_type=jnp.float32) * dis


def _fin_body(aa_ref, ab_ref, hs_ref, da_ref, db_ref, b2_ref, o_ref):
    dis = _dis(da_ref, db_ref)
    o_ref[...] = dis * (aa_ref[...] + ab_ref[...] + hs_ref[...]) + b2_ref[...]


def _row_spec(width):
    return pl.BlockSpec((RB, width), lambda i: (i, 0))


def _full_spec(shape):
    return pl.BlockSpec(shape, lambda i: (0,) * len(shape))


_l1_call = pl.pallas_call(
    _l1_body,
    out_shape=jax.ShapeDtypeStruct((NP, LATENT), jnp.float32),
    grid=(GRID,),
    in_specs=[
        _row_spec(LATENT), _row_spec(MAT), _row_spec(LATENT), _row_spec(LATENT),
        _full_spec((LATENT, LATENT)), _full_spec((MAT, LATENT)),
    ],
    out_specs=_row_spec(LATENT),
)

_l2_call = pl.pallas_call(
    _l2_body,
    out_shape=jax.ShapeDtypeStruct((NP, OUT), jnp.float32),
    grid=(GRID,),
    in_specs=[
        _row_spec(LATENT), _row_spec(LATENT), _row_spec(LATENT),
        _row_spec(LATENT), _row_spec(LATENT),
        _full_spec((1, LATENT)), _full_spec((LATENT, OUT)),
    ],
    out_specs=_row_spec(OUT),
)

_fin_call = pl.pallas_call(
    _fin_body,
    out_shape=jax.ShapeDtypeStruct((NP, OUT), jnp.float32),
    grid=(GRID,),
    in_specs=[
        _row_spec(OUT), _row_spec(OUT), _row_spec(OUT),
        _row_spec(LATENT), _row_spec(LATENT),
        _full_spec((1, OUT)),
    ],
    out_specs=_row_spec(OUT),
)


def kernel(z, edge_index, material_params, W1, b1, W2, b2):
    src = edge_index[0].astype(jnp.int32)
    dst = edge_index[1].astype(jnp.int32)
    dst_d = dst.reshape(NW, CHUNKS_D, KD)
    pad = jnp.full((E_PAD - E,), N, jnp.int32)
    cnt0 = CEF if FAST_CID == 0 else CES
    cnt1 = CEF if FAST_CID == 1 else CES

    def _split(flat):
        ch = flat.reshape(-1, KE)
        blk0 = ch[:NS * cnt0].reshape(NS, cnt0, KE)
        blk1 = ch[NS * cnt0:].reshape(NS, cnt1, KE)
        cem = max(cnt0, cnt1)
        fill = jnp.full((NS, cem, KE), N, jnp.int32)
        blk0 = fill.at[:, :cnt0].set(blk0)
        blk1 = fill.at[:, :cnt1].set(blk1)
        return jnp.concatenate([blk0, blk1], axis=0)

    srcp = _split(jnp.concatenate([src, pad]))
    dstp = _split(jnp.concatenate([dst, pad]))
    eidx = jnp.stack([srcp, dstp], axis=2)
    zp = jnp.pad(z, ((0, NP - N), (0, 0)))
    mpp = jnp.pad(material_params, ((0, NP - N), (0, 0)))
    zeros128 = jnp.zeros((NP, LATENT), jnp.float32)
    ones128 = jnp.ones((KD, LATENT), jnp.float32)

    deg = _deg_kernel(dst_d, ones128, zeros128)
    da, db = deg[0], deg[1]

    hs1 = _l1_call(zp, mpp, da, db, W1[:LATENT], W1[LATENT:])
    acc1 = _edge_kernel(hs1, eidx, zeros128)
    hs2 = _l2_call(acc1[0], acc1[1], hs1, da, db, b1.reshape(1, LATENT), W2)
    acc2 = _edge_kernel(hs2, eidx, zeros128)
    out = _fin_call(acc2[0], acc2[1], hs2, da, db, b2.reshape(1, OUT))
    return out[:N]


# speedup vs baseline: 1.1874x; 1.1874x over previous
import functools

import jax
import jax.numpy as jnp
from jax import lax
from jax.experimental import pallas as pl
from jax.experimental.pallas import tpu as pltpu
from jax.experimental.pallas import tpu_sc as plsc

N = 10000
NP = 10240
E = 320000
LATENT = 128
MAT = 16
OUT = 128

NC = 2
NS = 16
NW = NC * NS
EPW = E // NW
KD = 80
CHUNKS_D = EPW // KD
KE = 112
FAST_CID = 0
CEF = 120
CES = 60
CE_TOT = CEF + CES
E_PAD = NS * CE_TOT * KE
ROWS_PER_TILE = NP // NS

_mesh = plsc.VectorSubcoreMesh(core_axis_name="c", subcore_axis_name="s",
                               num_cores=NC, num_subcores=NS)


@functools.partial(
    pl.kernel,
    out_type=jax.ShapeDtypeStruct((NC, NP, LATENT), jnp.float32),
    mesh=_mesh,
    scratch_types=[
        pltpu.VMEM((CHUNKS_D, KD), jnp.int32),
        pltpu.VMEM((KD, LATENT), jnp.float32),
        pltpu.VMEM_SHARED((NP, LATENT), jnp.float32),
    ],
)
def _deg_kernel(dst_hbm, ones_hbm, zeros_hbm, out_hbm, idx_v, ones_v, acc_sh):
    cid = lax.axis_index("c")
    sid = lax.axis_index("s")
    wid = cid * NS + sid
    pltpu.sync_copy(dst_hbm.at[wid], idx_v)
    pltpu.sync_copy(ones_hbm, ones_v)
    pltpu.sync_copy(
        zeros_hbm.at[pl.ds(sid * ROWS_PER_TILE, ROWS_PER_TILE)],
        acc_sh.at[pl.ds(sid * ROWS_PER_TILE, ROWS_PER_TILE)],
    )
    plsc.subcore_barrier()

    def body(c, carry):
        pltpu.sync_copy(ones_v, acc_sh.at[idx_v.at[c]], add=True)
        return carry

    lax.fori_loop(0, CHUNKS_D, body, 0)
    plsc.subcore_barrier()
    pltpu.sync_copy(
        acc_sh.at[pl.ds(sid * ROWS_PER_TILE, ROWS_PER_TILE)],
        out_hbm.at[cid, pl.ds(sid * ROWS_PER_TILE, ROWS_PER_TILE)],
    )


@functools.partial(
    pl.kernel,
    out_type=jax.ShapeDtypeStruct((NC, NP, LATENT), jnp.float32),
    mesh=_mesh,
    scratch_types=[
        pltpu.VMEM((5, 2, KE), jnp.int32),
        pltpu.VMEM((3, KE, LATENT), jnp.float32),
        pltpu.VMEM_SHARED((NP, LATENT), jnp.float32),
    ] + [pltpu.SemaphoreType.DMA] * 11,
)
def _edge_kernel(hs_hbm, idx_hbm, zeros_hbm, out_hbm, idx_v, rows_v, acc_sh,
                 isem0, isem1, isem2, isem3, isem4,
                 gsem0, gsem1, gsem2, ssem0, ssem1, ssem2):
    cid = lax.axis_index("c")
    sid = lax.axis_index("s")
    wid = cid * NS + sid
    cnt = jnp.where(cid == FAST_CID, CEF, CES)
    isems = (isem0, isem1, isem2, isem3, isem4)
    gsems = (gsem0, gsem1, gsem2)
    ssems = (ssem0, ssem1, ssem2)

    for i in range(5):
        pltpu.async_copy(idx_hbm.at[wid, i], idx_v.at[i], isems[i])
    pltpu.sync_copy(
        zeros_hbm.at[pl.ds(sid * ROWS_PER_TILE, ROWS_PER_TILE)],
        acc_sh.at[pl.ds(sid * ROWS_PER_TILE, ROWS_PER_TILE)],
    )
    plsc.subcore_barrier()
    for c0 in (0, 1):
        pltpu.make_async_copy(
            idx_hbm.at[wid, c0], idx_v.at[c0], isems[c0]).wait()
        pltpu.async_copy(hs_hbm.at[idx_v.at[c0, 0]], rows_v.at[c0],
                         gsems[c0])

    NSTEP = 15

    def body(g, carry):
        base = g * NSTEP
        for st in range(NSTEP):
            r = st % 3
            rm1 = (st - 1) % 3
            rp2 = (st + 2) % 3
            i = st % 5
            im1 = (st - 1) % 5
            ip2 = (st + 2) % 5
            ip4 = (st + 4) % 5
            c = base + st

            pltpu.make_async_copy(
                hs_hbm.at[idx_v.at[i, 0]], rows_v.at[r], gsems[r]).wait()
            pltpu.async_copy(
                rows_v.at[r], acc_sh.at[idx_v.at[i, 1]], ssems[r], add=True)

            @pl.when(c >= 1)
            def _():
                pltpu.make_async_copy(
                    rows_v.at[rm1], acc_sh.at[idx_v.at[im1, 1]],
                    ssems[rm1]).wait()

            @pl.when((c >= 1) & (c + 4 < cnt))
            def _():
                pltpu.async_copy(
                    idx_hbm.at[wid, c + 4], idx_v.at[ip4], isems[ip4])

            @pl.when(c + 2 < cnt)
            def _():
                pltpu.make_async_copy(
                    idx_hbm.at[wid, c + 2], idx_v.at[ip2], isems[ip2]).wait()
                pltpu.async_copy(
                    hs_hbm.at[idx_v.at[ip2, 0]], rows_v.at[rp2], gsems[rp2])
        return carry

    lax.fori_loop(0, cnt // NSTEP, body, 0)
    lr = (CEF - 1) % 3
    li = (CEF - 1) % 5
    pltpu.make_async_copy(
        rows_v.at[lr], acc_sh.at[idx_v.at[li, 1]], ssems[lr]).wait()
    plsc.subcore_barrier()
    pltpu.sync_copy(
        acc_sh.at[pl.ds(sid * ROWS_PER_TILE, ROWS_PER_TILE)],
        out_hbm.at[cid, pl.ds(sid * ROWS_PER_TILE, ROWS_PER_TILE)],
    )


RB = 1024
GRID = NP // RB


def _dis(da_ref, db_ref):
    deg = 1.0 + da_ref[:, 0:1] + db_ref[:, 0:1]
    return lax.rsqrt(deg)


def _l1_body(z_ref, mp_ref, da_ref, db_ref, w1z_ref, w1m_ref, o_ref):
    h = jnp.dot(z_ref[...], w1z_ref[...], preferred_element_type=jnp.float32)
    h = h + jnp.dot(mp_ref[...], w1m_ref[...],
                    preferred_element_type=jnp.float32)
    o_ref[...] = h * _dis(da_ref, db_ref)


def _l2_body(aa_ref, ab_ref, hs_ref, da_ref, db_ref, b1_ref, w2_ref, o_ref):
    dis = _dis(da_ref, db_ref)
    x = dis * (aa_ref[...] + ab_ref[...] + hs_ref[...]) + b1_ref[...]
    x = jnp.maximum(x, 0.0)
    o_ref[...] = jnp.dot(x, w2_ref[...],
                         preferred_element_type=jnp.float32) * dis


def _fin_body(aa_ref, ab_ref, hs_ref, da_ref, db_ref, b2_ref, o_ref):
    dis = _dis(da_ref, db_ref)
    o_ref[...] = dis * (aa_ref[...] + ab_ref[...] + hs_ref[...]) + b2_ref[...]


def _row_spec(width):
    return pl.BlockSpec((RB, width), lambda i: (i, 0))


def _full_spec(shape):
    return pl.BlockSpec(shape, lambda i: (0,) * len(shape))


_l1_call = pl.pallas_call(
    _l1_body,
    out_shape=jax.ShapeDtypeStruct((NP, LATENT), jnp.float32),
    grid=(GRID,),
    in_specs=[
        _row_spec(LATENT), _row_spec(MAT), _row_spec(LATENT), _row_spec(LATENT),
        _full_spec((LATENT, LATENT)), _full_spec((MAT, LATENT)),
    ],
    out_specs=_row_spec(LATENT),
)

_l2_call = pl.pallas_call(
    _l2_body,
    out_shape=jax.ShapeDtypeStruct((NP, OUT), jnp.float32),
    grid=(GRID,),
    in_specs=[
        _row_spec(LATENT), _row_spec(LATENT), _row_spec(LATENT),
        _row_spec(LATENT), _row_spec(LATENT),
        _full_spec((1, LATENT)), _full_spec((LATENT, OUT)),
    ],
    out_specs=_row_spec(OUT),
)

_fin_call = pl.pallas_call(
    _fin_body,
    out_shape=jax.ShapeDtypeStruct((NP, OUT), jnp.float32),
    grid=(GRID,),
    in_specs=[
        _row_spec(OUT), _row_spec(OUT), _row_spec(OUT),
        _row_spec(LATENT), _row_spec(LATENT),
        _full_spec((1, OUT)),
    ],
    out_specs=_row_spec(OUT),
)


def kernel(z, edge_index, material_params, W1, b1, W2, b2):
    src = edge_index[0].astype(jnp.int32)
    dst = edge_index[1].astype(jnp.int32)
    dst_d = dst.reshape(NW, CHUNKS_D, KD)
    pad = jnp.full((E_PAD - E,), N, jnp.int32)
    cnt0 = CEF if FAST_CID == 0 else CES
    cnt1 = CEF if FAST_CID == 1 else CES

    def _split(flat):
        ch = flat.reshape(-1, KE)
        blk0 = ch[:NS * cnt0].reshape(NS, cnt0, KE)
        blk1 = ch[NS * cnt0:].reshape(NS, cnt1, KE)
        cem = max(cnt0, cnt1)
        fill = jnp.full((NS, cem, KE), N, jnp.int32)
        blk0 = fill.at[:, :cnt0].set(blk0)
        blk1 = fill.at[:, :cnt1].set(blk1)
        return jnp.concatenate([blk0, blk1], axis=0)

    srcp = _split(jnp.concatenate([src, pad]))
    dstp = _split(jnp.concatenate([dst, pad]))
    eidx = jnp.stack([srcp, dstp], axis=2)
    zp = jnp.pad(z, ((0, NP - N), (0, 0)))
    mpp = jnp.pad(material_params, ((0, NP - N), (0, 0)))
    zeros128 = jnp.zeros((NP, LATENT), jnp.float32)
    ones128 = jnp.ones((KD, LATENT), jnp.float32)

    deg = _deg_kernel(dst_d, ones128, zeros128)
    da, db = deg[0], deg[1]

    hs1 = _l1_call(zp, mpp, da, db, W1[:LATENT], W1[LATENT:])
    acc1 = _edge_kernel(hs1, eidx, zeros128)
    hs2 = _l2_call(acc1[0], acc1[1], hs1, da, db, b1.reshape(1, LATENT), W2)
    acc2 = _edge_kernel(hs2, eidx, zeros128)
    out = _fin_call(acc2[0], acc2[1], hs2, da, db, b2.reshape(1, OUT))
    return out[:N]


# speedup vs baseline: 1.2291x; 1.0351x over previous
import functools

import jax
import jax.numpy as jnp
from jax import lax
from jax.experimental import pallas as pl
from jax.experimental.pallas import tpu as pltpu
from jax.experimental.pallas import tpu_sc as plsc

N = 10000
NP = 10240
E = 320000
LATENT = 128
MAT = 16
OUT = 128

NC = 2
NS = 16
NW = NC * NS
EPW = E // NW
KD = 80
CHUNKS_D = EPW // KD
KE = 112
FAST_CID = 0
CEF = 135
CES = 45
CE_TOT = CEF + CES
E_PAD = NS * CE_TOT * KE
ROWS_PER_TILE = NP // NS

_mesh = plsc.VectorSubcoreMesh(core_axis_name="c", subcore_axis_name="s",
                               num_cores=NC, num_subcores=NS)


@functools.partial(
    pl.kernel,
    out_type=jax.ShapeDtypeStruct((NC, NP, LATENT), jnp.float32),
    mesh=_mesh,
    scratch_types=[
        pltpu.VMEM((CHUNKS_D, KD), jnp.int32),
        pltpu.VMEM((KD, LATENT), jnp.float32),
        pltpu.VMEM_SHARED((NP, LATENT), jnp.float32),
    ],
)
def _deg_kernel(dst_hbm, ones_hbm, zeros_hbm, out_hbm, idx_v, ones_v, acc_sh):
    cid = lax.axis_index("c")
    sid = lax.axis_index("s")
    wid = cid * NS + sid
    pltpu.sync_copy(dst_hbm.at[wid], idx_v)
    pltpu.sync_copy(ones_hbm, ones_v)
    pltpu.sync_copy(
        zeros_hbm.at[pl.ds(sid * ROWS_PER_TILE, ROWS_PER_TILE)],
        acc_sh.at[pl.ds(sid * ROWS_PER_TILE, ROWS_PER_TILE)],
    )
    plsc.subcore_barrier()

    def body(c, carry):
        pltpu.sync_copy(ones_v, acc_sh.at[idx_v.at[c]], add=True)
        return carry

    lax.fori_loop(0, CHUNKS_D, body, 0)
    plsc.subcore_barrier()
    pltpu.sync_copy(
        acc_sh.at[pl.ds(sid * ROWS_PER_TILE, ROWS_PER_TILE)],
        out_hbm.at[cid, pl.ds(sid * ROWS_PER_TILE, ROWS_PER_TILE)],
    )


@functools.partial(
    pl.kernel,
    out_type=jax.ShapeDtypeStruct((NC, NP, LATENT), jnp.float32),
    mesh=_mesh,
    scratch_types=[
        pltpu.VMEM((5, 2, KE), jnp.int32),
        pltpu.VMEM((3, KE, LATENT), jnp.float32),
        pltpu.VMEM_SHARED((NP, LATENT), jnp.float32),
    ] + [pltpu.SemaphoreType.DMA] * 11,
)
def _edge_kernel(hs_hbm, idx_hbm, zeros_hbm, out_hbm, idx_v, rows_v, acc_sh,
                 isem0, isem1, isem2, isem3, isem4,
                 gsem0, gsem1, gsem2, ssem0, ssem1, ssem2):
    cid = lax.axis_index("c")
    sid = lax.axis_index("s")
    wid = cid * NS + sid
    cnt = jnp.where(cid == FAST_CID, CEF, CES)
    isems = (isem0, isem1, isem2, isem3, isem4)
    gsems = (gsem0, gsem1, gsem2)
    ssems = (ssem0, ssem1, ssem2)

    for i in range(5):
        pltpu.async_copy(idx_hbm.at[wid, i], idx_v.at[i], isems[i])
    pltpu.sync_copy(
        zeros_hbm.at[pl.ds(sid * ROWS_PER_TILE, ROWS_PER_TILE)],
        acc_sh.at[pl.ds(sid * ROWS_PER_TILE, ROWS_PER_TILE)],
    )
    plsc.subcore_barrier()
    for c0 in (0, 1):
        pltpu.make_async_copy(
            idx_hbm.at[wid, c0], idx_v.at[c0], isems[c0]).wait()
        pltpu.async_copy(hs_hbm.at[idx_v.at[c0, 0]], rows_v.at[c0],
                         gsems[c0])

    NSTEP = 15

    def body(g, carry):
        base = g * NSTEP
        for st in range(NSTEP):
            r = st % 3
            rm1 = (st - 1) % 3
            rp2 = (st + 2) % 3
            i = st % 5
            im1 = (st - 1) % 5
            ip2 = (st + 2) % 5
            ip4 = (st + 4) % 5
            c = base + st

            pltpu.make_async_copy(
                hs_hbm.at[idx_v.at[i, 0]], rows_v.at[r], gsems[r]).wait()
            pltpu.async_copy(
                rows_v.at[r], acc_sh.at[idx_v.at[i, 1]], ssems[r], add=True)

            @pl.when(c >= 1)
            def _():
                pltpu.make_async_copy(
                    rows_v.at[rm1], acc_sh.at[idx_v.at[im1, 1]],
                    ssems[rm1]).wait()

            @pl.when((c >= 1) & (c + 4 < cnt))
            def _():
                pltpu.async_copy(
                    idx_hbm.at[wid, c + 4], idx_v.at[ip4], isems[ip4])

            @pl.when(c + 2 < cnt)
            def _():
                pltpu.make_async_copy(
                    idx_hbm.at[wid, c + 2], idx_v.at[ip2], isems[ip2]).wait()
                pltpu.async_copy(
                    hs_hbm.at[idx_v.at[ip2, 0]], rows_v.at[rp2], gsems[rp2])
        return carry

    lax.fori_loop(0, cnt // NSTEP, body, 0)
    lr = (CEF - 1) % 3
    li = (CEF - 1) % 5
    pltpu.make_async_copy(
        rows_v.at[lr], acc_sh.at[idx_v.at[li, 1]], ssems[lr]).wait()
    plsc.subcore_barrier()
    pltpu.sync_copy(
        acc_sh.at[pl.ds(sid * ROWS_PER_TILE, ROWS_PER_TILE)],
        out_hbm.at[cid, pl.ds(sid * ROWS_PER_TILE, ROWS_PER_TILE)],
    )


RB = 1024
GRID = NP // RB


def _dis(da_ref, db_ref):
    deg = 1.0 + da_ref[:, 0:1] + db_ref[:, 0:1]
    return lax.rsqrt(deg)


def _l1_body(z_ref, mp_ref, da_ref, db_ref, w1z_ref, w1m_ref, o_ref):
    h = jnp.dot(z_ref[...], w1z_ref[...], preferred_element_type=jnp.float32)
    h = h + jnp.dot(mp_ref[...], w1m_ref[...],
                    preferred_element_type=jnp.float32)
    o_ref[...] = h * _dis(da_ref, db_ref)


def _l2_body(aa_ref, ab_ref, hs_ref, da_ref, db_ref, b1_ref, w2_ref, o_ref):
    dis = _dis(da_ref, db_ref)
    x = dis * (aa_ref[...] + ab_ref[...] + hs_ref[...]) + b1_ref[...]
    x = jnp.maximum(x, 0.0)
    o_ref[...] = jnp.dot(x, w2_ref[...],
                         preferred_element_type=jnp.float32) * dis


def _fin_body(aa_ref, ab_ref, hs_ref, da_ref, db_ref, b2_ref, o_ref):
    dis = _dis(da_ref, db_ref)
    o_ref[...] = dis * (aa_ref[...] + ab_ref[...] + hs_ref[...]) + b2_ref[...]


def _row_spec(width):
    return pl.BlockSpec((RB, width), lambda i: (i, 0))


def _full_spec(shape):
    return pl.BlockSpec(shape, lambda i: (0,) * len(shape))


_l1_call = pl.pallas_call(
    _l1_body,
    out_shape=jax.ShapeDtypeStruct((NP, LATENT), jnp.float32),
    grid=(GRID,),
    in_specs=[
        _row_spec(LATENT), _row_spec(MAT), _row_spec(LATENT), _row_spec(LATENT),
        _full_spec((LATENT, LATENT)), _full_spec((MAT, LATENT)),
    ],
    out_specs=_row_spec(LATENT),
)

_l2_call = pl.pallas_call(
    _l2_body,
    out_shape=jax.ShapeDtypeStruct((NP, OUT), jnp.float32),
    grid=(GRID,),
    in_specs=[
        _row_spec(LATENT), _row_spec(LATENT), _row_spec(LATENT),
        _row_spec(LATENT), _row_spec(LATENT),
        _full_spec((1, LATENT)), _full_spec((LATENT, OUT)),
    ],
    out_specs=_row_spec(OUT),
)

_fin_call = pl.pallas_call(
    _fin_body,
    out_shape=jax.ShapeDtypeStruct((NP, OUT), jnp.float32),
    grid=(GRID,),
    in_specs=[
        _row_spec(OUT), _row_spec(OUT), _row_spec(OUT),
        _row_spec(LATENT), _row_spec(LATENT),
        _full_spec((1, OUT)),
    ],
    out_specs=_row_spec(OUT),
)


def kernel(z, edge_index, material_params, W1, b1, W2, b2):
    src = edge_index[0].astype(jnp.int32)
    dst = edge_index[1].astype(jnp.int32)
    dst_d = dst.reshape(NW, CHUNKS_D, KD)
    pad = jnp.full((E_PAD - E,), N, jnp.int32)
    cnt0 = CEF if FAST_CID == 0 else CES
    cnt1 = CEF if FAST_CID == 1 else CES

    def _split(flat):
        ch = flat.reshape(-1, KE)
        blk0 = ch[:NS * cnt0].reshape(NS, cnt0, KE)
        blk1 = ch[NS * cnt0:].reshape(NS, cnt1, KE)
        cem = max(cnt0, cnt1)
        fill = jnp.full((NS, cem, KE), N, jnp.int32)
        blk0 = fill.at[:, :cnt0].set(blk0)
        blk1 = fill.at[:, :cnt1].set(blk1)
        return jnp.concatenate([blk0, blk1], axis=0)

    srcp = _split(jnp.concatenate([src, pad]))
    dstp = _split(jnp.concatenate([dst, pad]))
    eidx = jnp.stack([srcp, dstp], axis=2)
    zp = jnp.pad(z, ((0, NP - N), (0, 0)))
    mpp = jnp.pad(material_params, ((0, NP - N), (0, 0)))
    zeros128 = jnp.zeros((NP, LATENT), jnp.float32)
    ones128 = jnp.ones((KD, LATENT), jnp.float32)

    deg = _deg_kernel(dst_d, ones128, zeros128)
    da, db = deg[0], deg[1]

    hs1 = _l1_call(zp, mpp, da, db, W1[:LATENT], W1[LATENT:])
    acc1 = _edge_kernel(hs1, eidx, zeros128)
    hs2 = _l2_call(acc1[0], acc1[1], hs1, da, db, b1.reshape(1, LATENT), W2)
    acc2 = _edge_kernel(hs2, eidx, zeros128)
    out = _fin_call(acc2[0], acc2[1], hs2, da, db, b2.reshape(1, OUT))
    return out[:N]
